# baseline (speedup 1.0000x reference)
import functools

import jax
import jax.numpy as jnp
from jax import lax
from jax.experimental import pallas as pl
from jax.experimental.pallas import tpu as pltpu
from jax.experimental.pallas import tpu_sc as plsc

N = 10000
E = 320000
D = 128
H = 128
HT = 64
T = 4
G = 200
NPG = 50
K = 25

NC = 2
NS = 16
NW = NC * NS

CE = 128
CH = 80
EP = NW * CH * CE
PADIDX = N

C0S = 96
C1S = 64
NPAD = N + 16
RPT = 640
NACC = NS * RPT


def _sc_edge_pass(table, srcp, dstp, feat, gather, c0=CH, c1=CH):
    mesh = plsc.VectorSubcoreMesh(core_axis_name="c", subcore_axis_name="s",
                                  num_cores=NC, num_subcores=NS)
    cmax = max(c0, c1)

    @functools.partial(
        pl.kernel,
        out_type=jax.ShapeDtypeStruct((NC, NACC, feat), jnp.float32),
        mesh=mesh,
        scratch_types=[
            pltpu.VMEM((cmax, CE), jnp.int32),
            pltpu.VMEM((cmax, CE), jnp.int32),
            pltpu.VMEM((CE, feat), jnp.float32),
            pltpu.VMEM_SHARED((NACC, feat), jnp.float32),
            pltpu.SemaphoreType.DMA,
            pltpu.SemaphoreType.DMA,
        ],
    )
    def kern(table_hbm, src_hbm, dst_hbm, out_hbm, sidx, didx, buf, acc,
             gsem, ssem):
        c = lax.axis_index("c")
        s = lax.axis_index("s")

        def mset(i, _):
            for jj in range(feat // 16):
                buf[i, pl.ds(jj * 16, 16)] = jnp.zeros((16,), jnp.float32)
            return 0

        lax.fori_loop(0, CE, mset, 0)

        def zstep(kk, _):
            pltpu.sync_copy(buf.at[pl.ds(0, CE)],
                            acc.at[pl.ds(s * RPT + kk * CE, CE)])
            return 0

        lax.fori_loop(0, RPT // CE, zstep, 0)

        if not gather:
            e0 = jnp.where(lax.iota(jnp.int32, 16) == 0,
                           jnp.float32(1), jnp.float32(0))

            def oset(i, _):
                buf[i, pl.ds(0, 16)] = e0
                return 0

            lax.fori_loop(0, CE, oset, 0)

        @pl.when(c == 0)
        def _():
            pltpu.sync_copy(src_hbm.at[pl.ds(s * c0, c0)],
                            sidx.at[pl.ds(0, c0)])
            pltpu.sync_copy(dst_hbm.at[pl.ds(s * c0, c0)],
                            didx.at[pl.ds(0, c0)])

        @pl.when(c == 1)
        def _():
            pltpu.sync_copy(src_hbm.at[pl.ds(NS * c0 + s * c1, c1)],
                            sidx.at[pl.ds(0, c1)])
            pltpu.sync_copy(dst_hbm.at[pl.ds(NS * c0 + s * c1, c1)],
                            didx.at[pl.ds(0, c1)])

        nch = jnp.where(c == 0, c0, c1)
        plsc.subcore_barrier()

        if gather:
            def step(i, _):
                pltpu.async_copy(
                    table_hbm.at[sidx.at[i]], buf.at[pl.ds(0, CE)],
                    gsem).wait()
                pltpu.sync_copy(buf.at[pl.ds(0, CE)], acc.at[didx.at[i]],
                                add=True)
                return 0

            lax.fori_loop(0, nch, step, 0)
        else:
            def step(i, _):
                pltpu.sync_copy(buf.at[pl.ds(0, CE)], acc.at[didx.at[i]],
                                add=True)
                return 0

            lax.fori_loop(0, nch, step, 0)
        plsc.subcore_barrier()

        def wout(kk, _):
            r0 = s * RPT + kk * CE
            pltpu.sync_copy(acc.at[pl.ds(r0, CE)], buf.at[pl.ds(0, CE)])
            pltpu.sync_copy(buf.at[pl.ds(0, CE)],
                            out_hbm.at[c, pl.ds(r0, CE)])
            return 0

        lax.fori_loop(0, RPT // CE, wout, 0)

    return kern(table, srcp, dstp)


def _tc(body, out_shape, *args):
    return pl.pallas_call(body, out_shape=out_shape)(*args)


def _tc1(x_ref, w1_ref, dacc_ref, dinv_ref, ut1_ref):
    deg = dacc_ref[0, :N, 0:1] + dacc_ref[1, :N, 0:1] + 1.0
    dinv = lax.rsqrt(deg)
    dinv_ref[...] = dinv
    u = jnp.dot(x_ref[...], w1_ref[...], preferred_element_type=jnp.float32)
    ut1_ref[...] = jnp.concatenate(
        [dinv * u, jnp.zeros((NPAD - N, H), jnp.float32)], axis=0)


def _tc2(macc_ref, ut_ref, dinv_ref, b_ref, w_ref, out_ref):
    dinv = dinv_ref[...]
    pre = dinv * (macc_ref[0, :N] + macc_ref[1, :N] + ut_ref[:N]) + b_ref[...]
    h = jnp.maximum(pre, 0.0)
    u = jnp.dot(h, w_ref[...], preferred_element_type=jnp.float32)
    out_ref[...] = jnp.concatenate(
        [dinv * u, jnp.zeros((NPAD - N, H), jnp.float32)], axis=0)


def _tc3a(macc_ref, ut_ref, dinv_ref, b_ref, p_ref, wt_ref, wl_ref,
          s_ref, u_ref):
    dinv = dinv_ref[...]
    pre = dinv * (macc_ref[0, :N] + macc_ref[1, :N] + ut_ref[:N]) + b_ref[...]
    h2 = jnp.maximum(pre, 0.0)
    p = p_ref[...]
    pn = p / jnp.sqrt(jnp.sum(p * p, axis=1, keepdims=True))
    s_ref[...] = jnp.dot(h2, pn.T, preferred_element_type=jnp.float32)
    wt = wt_ref[...]
    wl = wl_ref[...]
    C = jnp.concatenate([jnp.dot(wt[t], wl[t],
                                 preferred_element_type=jnp.float32)
                         for t in range(T)], axis=1)
    u_ref[...] = jnp.dot(h2, C, preferred_element_type=jnp.float32)


def _tc3b(s_ref, u_ref, ktab_ref, m_ref, a_ref):
    S = s_ref[...]
    U = u_ref[...]
    gid = lax.broadcasted_iota(jnp.int32, (N, G), 1)
    ng = lax.broadcasted_iota(jnp.int32, (N, G), 0) // NPG
    OG = (gid == ng).astype(jnp.bfloat16)
    pj = lax.broadcasted_iota(jnp.int32, (N, NPG), 1)
    pn = lax.broadcasted_iota(jnp.int32, (N, NPG), 0) % NPG
    samepos = pj == pn
    rowi = lax.broadcasted_iota(jnp.int32, (NPG, G), 0)
    ms = []
    for t in range(T):
        P = jnp.where(samepos, S[:, t:t + 1], 0.0)
        A = lax.dot_general(P, OG, (((0,), (0,)), ((), ())),
                            preferred_element_type=jnp.float32)
        rank = jnp.zeros((NPG, G), jnp.float32)
        for j in range(NPG):
            sj = A[j:j + 1, :]
            rank = rank + (sj > A).astype(jnp.float32)
            rank = rank + ((sj == A) & (rowi > j)).astype(jnp.float32)
        mem2 = (rank < K).astype(jnp.bfloat16)
        Y = lax.dot_general(OG, mem2, (((1,), (1,)), ((), ())),
                            preferred_element_type=jnp.float32)
        ms.append(jnp.sum(jnp.where(samepos, Y, 0.0),
                          axis=1, keepdims=True))
    m = jnp.concatenate(ms, axis=1)
    m_ref[...] = m
    a_ref[...] = m * jnp.tanh(S) * U
    ktab_ref[...] = jnp.concatenate(
        [jnp.concatenate([m, jnp.zeros((N, H - T), jnp.float32)], axis=1),
         jnp.zeros((NPAD - N, H), jnp.float32)], axis=0)


def _tc4(cacc_ref, m_ref, a_ref, qtab_ref, q_ref, r_ref):
    m = m_ref[...]
    cnt = cacc_ref[0, :N, :T] + cacc_ref[1, :N, :T]
    dinv_t = lax.rsqrt(1.0 + m * cnt)
    q = dinv_t * a_ref[...]
    r = dinv_t * m
    q_ref[...] = q
    r_ref[...] = r
    qtab_ref[...] = jnp.concatenate(
        [jnp.concatenate([q, jnp.zeros((N, H - T), jnp.float32)], axis=1),
         jnp.zeros((NPAD - N, H), jnp.float32)], axis=0)


def _tc5(gacc_ref, q_ref, r_ref, bt_ref, wl_ref, bl_ref, out_ref):
    gsum = gacc_ref[0, :N, :T] + gacc_ref[1, :N, :T]
    contrib = r_ref[...] * (gsum + q_ref[...])
    gid = lax.broadcasted_iota(jnp.int32, (N, G), 1)
    ng = lax.broadcasted_iota(jnp.int32, (N, G), 0) // NPG
    OG = (gid == ng).astype(jnp.bfloat16)
    pooled = lax.dot_general(OG, contrib, (((0,), (0,)), ((), ())),
                             preferred_element_type=jnp.float32)
    bt2 = bt_ref[...]
    wl2 = wl_ref[...].reshape(T, HT)
    bl2 = bl_ref[...]
    const = K * jnp.sum(bt2 * wl2, axis=1, keepdims=True) + bl2
    out_ref[...] = jax.nn.sigmoid(pooled + const.reshape(1, T))


def kernel(x, edge_index, num_atoms, W1, b1, W2, b2, p, Wt, bt, Wl, bl):
    del num_atoms
    src = edge_index[0]
    dst = edge_index[1]
    pad = jnp.full((EP - E,), PADIDX, jnp.int32)
    srcp = jnp.concatenate([src, pad]).reshape(NW * CH, CE)
    dstp = jnp.concatenate([dst, pad]).reshape(NW * CH, CE)

    ztab = jnp.zeros((NPAD, H), jnp.float32)

    dacc = _sc_edge_pass(ztab, srcp, dstp, H, gather=False)

    dinv, ut1 = _tc(
        _tc1,
        (jax.ShapeDtypeStruct((N, 1), jnp.float32),
         jax.ShapeDtypeStruct((NPAD, H), jnp.float32)),
        x, W1, dacc)
    macc1 = _sc_edge_pass(ut1, srcp, dstp, H, gather=True, c0=C0S, c1=C1S)

    ut2 = _tc(_tc2, jax.ShapeDtypeStruct((NPAD, H), jnp.float32),
              macc1, ut1, dinv, b1, W2)
    macc2 = _sc_edge_pass(ut2, srcp, dstp, H, gather=True, c0=C0S, c1=C1S)

    S, U = _tc(
        _tc3a,
        (jax.ShapeDtypeStruct((N, T), jnp.float32),
         jax.ShapeDtypeStruct((N, T), jnp.float32)),
        macc2, ut2, dinv, b2, p, Wt, Wl)
    ktab, m, a = _tc(
        _tc3b,
        (jax.ShapeDtypeStruct((NPAD, H), jnp.float32),
         jax.ShapeDtypeStruct((N, T), jnp.float32),
         jax.ShapeDtypeStruct((N, T), jnp.float32)),
        S, U)

    cacc = _sc_edge_pass(ktab, srcp, dstp, H, gather=True, c0=C0S, c1=C1S)

    qtab, q, r = _tc(
        _tc4,
        (jax.ShapeDtypeStruct((NPAD, H), jnp.float32),
         jax.ShapeDtypeStruct((N, T), jnp.float32),
         jax.ShapeDtypeStruct((N, T), jnp.float32)),
        cacc, m, a)

    gacc = _sc_edge_pass(qtab, srcp, dstp, H, gather=True, c0=C0S, c1=C1S)

    out = _tc(_tc5, jax.ShapeDtypeStruct((G, T), jnp.float32),
              gacc, q, r, bt, Wl, bl)
    return out.reshape(-1)[:, None]


# speedup vs baseline: 1.1358x; 1.1358x over previous
import functools

import jax
import jax.numpy as jnp
from jax import lax
from jax.experimental import pallas as pl
from jax.experimental.pallas import tpu as pltpu
from jax.experimental.pallas import tpu_sc as plsc

N = 10000
E = 320000
D = 128
H = 128
HT = 64
T = 4
G = 200
NPG = 50
K = 25

NC = 2
NS = 16
NW = NC * NS

CE = 128
CH = 80
EP = NW * CH * CE
PADIDX = N

C0S = 128
C1S = 32
NPAD = N + 16
RPT = 640
NACC = NS * RPT


def _sc_edge_pass(table, srcp, dstp, feat, gather, c0=CH, c1=CH):
    mesh = plsc.VectorSubcoreMesh(core_axis_name="c", subcore_axis_name="s",
                                  num_cores=NC, num_subcores=NS)
    cmax = max(c0, c1)

    @functools.partial(
        pl.kernel,
        out_type=jax.ShapeDtypeStruct((NC, NACC, feat), jnp.float32),
        mesh=mesh,
        scratch_types=[
            pltpu.VMEM((cmax, CE), jnp.int32),
            pltpu.VMEM((cmax, CE), jnp.int32),
            pltpu.VMEM((CE, feat), jnp.float32),
            pltpu.VMEM_SHARED((NACC, feat), jnp.float32),
            pltpu.SemaphoreType.DMA,
            pltpu.SemaphoreType.DMA,
        ],
    )
    def kern(table_hbm, src_hbm, dst_hbm, out_hbm, sidx, didx, buf, acc,
             gsem, ssem):
        c = lax.axis_index("c")
        s = lax.axis_index("s")

        def mset(i, _):
            for jj in range(feat // 16):
                buf[i, pl.ds(jj * 16, 16)] = jnp.zeros((16,), jnp.float32)
            return 0

        lax.fori_loop(0, CE, mset, 0)

        def zstep(kk, _):
            pltpu.sync_copy(buf.at[pl.ds(0, CE)],
                            acc.at[pl.ds(s * RPT + kk * CE, CE)])
            return 0

        lax.fori_loop(0, RPT // CE, zstep, 0)

        if not gather:
            e0 = jnp.where(lax.iota(jnp.int32, 16) == 0,
                           jnp.float32(1), jnp.float32(0))

            def oset(i, _):
                buf[i, pl.ds(0, 16)] = e0
                return 0

            lax.fori_loop(0, CE, oset, 0)

        @pl.when(c == 0)
        def _():
            pltpu.sync_copy(src_hbm.at[pl.ds(s * c0, c0)],
                            sidx.at[pl.ds(0, c0)])
            pltpu.sync_copy(dst_hbm.at[pl.ds(s * c0, c0)],
                            didx.at[pl.ds(0, c0)])

        @pl.when(c == 1)
        def _():
            pltpu.sync_copy(src_hbm.at[pl.ds(NS * c0 + s * c1, c1)],
                            sidx.at[pl.ds(0, c1)])
            pltpu.sync_copy(dst_hbm.at[pl.ds(NS * c0 + s * c1, c1)],
                            didx.at[pl.ds(0, c1)])

        nch = jnp.where(c == 0, c0, c1)
        plsc.subcore_barrier()

        if gather:
            def step(i, _):
                pltpu.async_copy(
                    table_hbm.at[sidx.at[i]], buf.at[pl.ds(0, CE)],
                    gsem).wait()
                pltpu.sync_copy(buf.at[pl.ds(0, CE)], acc.at[didx.at[i]],
                                add=True)
                return 0

            lax.fori_loop(0, nch, step, 0)
        else:
            def step(i, _):
                pltpu.sync_copy(buf.at[pl.ds(0, CE)], acc.at[didx.at[i]],
                                add=True)
                return 0

            lax.fori_loop(0, nch, step, 0)
        plsc.subcore_barrier()

        def wout(kk, _):
            r0 = s * RPT + kk * CE
            pltpu.sync_copy(acc.at[pl.ds(r0, CE)], buf.at[pl.ds(0, CE)])
            pltpu.sync_copy(buf.at[pl.ds(0, CE)],
                            out_hbm.at[c, pl.ds(r0, CE)])
            return 0

        lax.fori_loop(0, RPT // CE, wout, 0)

    return kern(table, srcp, dstp)


def _tc(body, out_shape, *args):
    return pl.pallas_call(body, out_shape=out_shape)(*args)


def _tc1(x_ref, w1_ref, dacc_ref, dinv_ref, ut1_ref):
    deg = dacc_ref[0, :N, 0:1] + dacc_ref[1, :N, 0:1] + 1.0
    dinv = lax.rsqrt(deg)
    dinv_ref[...] = dinv
    u = jnp.dot(x_ref[...], w1_ref[...], preferred_element_type=jnp.float32)
    ut1_ref[...] = jnp.concatenate(
        [dinv * u, jnp.zeros((NPAD - N, H), jnp.float32)], axis=0)


def _tc2(macc_ref, ut_ref, dinv_ref, b_ref, w_ref, out_ref):
    dinv = dinv_ref[...]
    pre = dinv * (macc_ref[0, :N] + macc_ref[1, :N] + ut_ref[:N]) + b_ref[...]
    h = jnp.maximum(pre, 0.0)
    u = jnp.dot(h, w_ref[...], preferred_element_type=jnp.float32)
    out_ref[...] = jnp.concatenate(
        [dinv * u, jnp.zeros((NPAD - N, H), jnp.float32)], axis=0)


def _tc3a(macc_ref, ut_ref, dinv_ref, b_ref, p_ref, wt_ref, wl_ref,
          s_ref, u_ref):
    dinv = dinv_ref[...]
    pre = dinv * (macc_ref[0, :N] + macc_ref[1, :N] + ut_ref[:N]) + b_ref[...]
    h2 = jnp.maximum(pre, 0.0)
    p = p_ref[...]
    pn = p / jnp.sqrt(jnp.sum(p * p, axis=1, keepdims=True))
    s_ref[...] = jnp.dot(h2, pn.T, preferred_element_type=jnp.float32)
    wt = wt_ref[...]
    wl = wl_ref[...]
    C = jnp.concatenate([jnp.dot(wt[t], wl[t],
                                 preferred_element_type=jnp.float32)
                         for t in range(T)], axis=1)
    u_ref[...] = jnp.dot(h2, C, preferred_element_type=jnp.float32)


def _tc3b(s_ref, u_ref, ktab_ref, m_ref, a_ref):
    S = s_ref[...]
    U = u_ref[...]
    gid = lax.broadcasted_iota(jnp.int32, (N, G), 1)
    ng = lax.broadcasted_iota(jnp.int32, (N, G), 0) // NPG
    OG = (gid == ng).astype(jnp.bfloat16)
    pj = lax.broadcasted_iota(jnp.int32, (N, NPG), 1)
    pn = lax.broadcasted_iota(jnp.int32, (N, NPG), 0) % NPG
    samepos = pj == pn
    rowi = lax.broadcasted_iota(jnp.int32, (NPG, G), 0)
    ms = []
    for t in range(T):
        P = jnp.where(samepos, S[:, t:t + 1], 0.0)
        A = lax.dot_general(P, OG, (((0,), (0,)), ((), ())),
                            preferred_element_type=jnp.float32)
        rank = jnp.zeros((NPG, G), jnp.float32)
        for j in range(NPG):
            sj = A[j:j + 1, :]
            rank = rank + (sj > A).astype(jnp.float32)
            rank = rank + ((sj == A) & (rowi > j)).astype(jnp.float32)
        mem2 = (rank < K).astype(jnp.bfloat16)
        Y = lax.dot_general(OG, mem2, (((1,), (1,)), ((), ())),
                            preferred_element_type=jnp.float32)
        ms.append(jnp.sum(jnp.where(samepos, Y, 0.0),
                          axis=1, keepdims=True))
    m = jnp.concatenate(ms, axis=1)
    m_ref[...] = m
    a_ref[...] = m * jnp.tanh(S) * U
    ktab_ref[...] = jnp.concatenate(
        [jnp.concatenate([m, jnp.zeros((N, H - T), jnp.float32)], axis=1),
         jnp.zeros((NPAD - N, H), jnp.float32)], axis=0)


def _tc4(cacc_ref, m_ref, a_ref, qtab_ref, q_ref, r_ref):
    m = m_ref[...]
    cnt = cacc_ref[0, :N, :T] + cacc_ref[1, :N, :T]
    dinv_t = lax.rsqrt(1.0 + m * cnt)
    q = dinv_t * a_ref[...]
    r = dinv_t * m
    q_ref[...] = q
    r_ref[...] = r
    qtab_ref[...] = jnp.concatenate(
        [jnp.concatenate([q, jnp.zeros((N, H - T), jnp.float32)], axis=1),
         jnp.zeros((NPAD - N, H), jnp.float32)], axis=0)


def _tc5(gacc_ref, q_ref, r_ref, bt_ref, wl_ref, bl_ref, out_ref):
    gsum = gacc_ref[0, :N, :T] + gacc_ref[1, :N, :T]
    contrib = r_ref[...] * (gsum + q_ref[...])
    gid = lax.broadcasted_iota(jnp.int32, (N, G), 1)
    ng = lax.broadcasted_iota(jnp.int32, (N, G), 0) // NPG
    OG = (gid == ng).astype(jnp.bfloat16)
    pooled = lax.dot_general(OG, contrib, (((0,), (0,)), ((), ())),
                             preferred_element_type=jnp.float32)
    bt2 = bt_ref[...]
    wl2 = wl_ref[...].reshape(T, HT)
    bl2 = bl_ref[...]
    const = K * jnp.sum(bt2 * wl2, axis=1, keepdims=True) + bl2
    out_ref[...] = jax.nn.sigmoid(pooled + const.reshape(1, T))


def kernel(x, edge_index, num_atoms, W1, b1, W2, b2, p, Wt, bt, Wl, bl):
    del num_atoms
    src = edge_index[0]
    dst = edge_index[1]
    pad = jnp.full((EP - E,), PADIDX, jnp.int32)
    srcp = jnp.concatenate([src, pad]).reshape(NW * CH, CE)
    dstp = jnp.concatenate([dst, pad]).reshape(NW * CH, CE)

    ztab = jnp.zeros((NPAD, H), jnp.float32)

    dacc = _sc_edge_pass(ztab, srcp, dstp, H, gather=False)

    dinv, ut1 = _tc(
        _tc1,
        (jax.ShapeDtypeStruct((N, 1), jnp.float32),
         jax.ShapeDtypeStruct((NPAD, H), jnp.float32)),
        x, W1, dacc)
    macc1 = _sc_edge_pass(ut1, srcp, dstp, H, gather=True, c0=C0S, c1=C1S)

    ut2 = _tc(_tc2, jax.ShapeDtypeStruct((NPAD, H), jnp.float32),
              macc1, ut1, dinv, b1, W2)
    macc2 = _sc_edge_pass(ut2, srcp, dstp, H, gather=True, c0=C0S, c1=C1S)

    S, U = _tc(
        _tc3a,
        (jax.ShapeDtypeStruct((N, T), jnp.float32),
         jax.ShapeDtypeStruct((N, T), jnp.float32)),
        macc2, ut2, dinv, b2, p, Wt, Wl)
    ktab, m, a = _tc(
        _tc3b,
        (jax.ShapeDtypeStruct((NPAD, H), jnp.float32),
         jax.ShapeDtypeStruct((N, T), jnp.float32),
         jax.ShapeDtypeStruct((N, T), jnp.float32)),
        S, U)

    cacc = _sc_edge_pass(ktab, srcp, dstp, H, gather=True, c0=C0S, c1=C1S)

    qtab, q, r = _tc(
        _tc4,
        (jax.ShapeDtypeStruct((NPAD, H), jnp.float32),
         jax.ShapeDtypeStruct((N, T), jnp.float32),
         jax.ShapeDtypeStruct((N, T), jnp.float32)),
        cacc, m, a)

    gacc = _sc_edge_pass(qtab, srcp, dstp, H, gather=True, c0=C0S, c1=C1S)

    out = _tc(_tc5, jax.ShapeDtypeStruct((G, T), jnp.float32),
              gacc, q, r, bt, Wl, bl)
    return out.reshape(-1)[:, None]
